# baseline (speedup 1.0000x reference)
import functools

import jax
import jax.numpy as jnp
from jax import lax
from jax.experimental import pallas as pl
from jax.experimental.pallas import tpu as pltpu
from jax.experimental.pallas import tpu_sc as plsc

_W = 1024
_NBUF = 16


@functools.lru_cache(maxsize=None)
def _build(B, V, D):
    info = plsc.get_sparse_core_info()
    NC, NS, L = info.num_cores, info.num_subcores, info.num_lanes
    NW = NC * NS
    FPT = D // NW
    n_full = V // _W
    mid = (V - n_full * _W) // 128 * 128
    tail = V - n_full * _W - mid
    n_grp = n_full // _NBUF
    assert FPT >= 1 and _W % 128 == 0 and n_full % _NBUF == 0 and B % L == 0
    mesh = plsc.VectorSubcoreMesh(core_axis_name="c", subcore_axis_name="s")

    @functools.partial(
        pl.kernel,
        mesh=mesh,
        out_type=jax.ShapeDtypeStruct((D, B), jnp.float32),
        scratch_types=[
            pltpu.VMEM((B + L,), jnp.int32),
            pltpu.VMEM((B + L,), jnp.int32),
            [pltpu.VMEM((FPT, _W), jnp.float32)] * _NBUF,
            pltpu.VMEM((FPT, max(tail, 1)), jnp.float32),
            pltpu.VMEM((FPT, B), jnp.float32),
            [pltpu.SemaphoreType.DMA] * _NBUF,
        ],
        compiler_params=pltpu.CompilerParams(needs_layout_passes=False),
    )
    def gather_kernel(embt_hbm, sidx_hbm, pos_hbm, tailt_hbm, outt_hbm,
                      sidx_v, pos_v, bufs, tbuf, outt_v, sems):
        wid = lax.axis_index("s") * NC + lax.axis_index("c")
        f0 = wid * FPT
        rows = pl.ds(f0, FPT)
        pltpu.sync_copy(sidx_hbm, sidx_v)
        pltpu.sync_copy(pos_hbm, pos_v)

        def proc_block(n0, n1, buf, j):
            def cond(carry):
                return carry[1]

            def step(carry):
                j, _ = carry
                v = sidx_v[pl.ds(j, L)]
                mask = v < n1
                local = v - n0
                p = pos_v[pl.ds(j, L)]
                for f in range(FPT):
                    fs = jnp.full((L,), 0, jnp.int32) + f
                    vals = plsc.load_gather(buf, [fs, local], mask=mask)
                    plsc.store_scatter(outt_v, [fs, p], vals, mask=mask)
                cnt = plsc.all_reduce_population_count(mask)[0]
                return j + cnt, cnt == L

            j, _ = lax.while_loop(cond, step, (j, True))
            return j

        def start(k, buf, sem):
            pltpu.async_copy(
                embt_hbm.at[rows, pl.ds(k * _W, _W)], buf, sem)

        def wait(k, buf, sem):
            pltpu.make_async_copy(
                embt_hbm.at[rows, pl.ds(k * _W, _W)], buf, sem).wait()

        for u in range(_NBUF):
            start(u, bufs[u], sems[u])

        def grp_body(i, j):
            k0 = _NBUF * i
            for u in range(_NBUF):
                k = k0 + u
                wait(k, bufs[u], sems[u])
                j = proc_block(k * _W, (k + 1) * _W, bufs[u], j)

                @pl.when(i < n_grp - 1)
                def _():
                    start(k + _NBUF, bufs[u], sems[u])

            return j

        j = lax.fori_loop(0, n_grp, grp_body, 0)

        if mid:
            mbuf = bufs[0].at[:, pl.ds(0, mid)]
            pltpu.sync_copy(embt_hbm.at[rows, pl.ds(n_full * _W, mid)], mbuf)
            j = proc_block(n_full * _W, n_full * _W + mid, mbuf, j)

        if tail:
            pltpu.sync_copy(tailt_hbm.at[rows, :], tbuf)
            proc_block(V - tail, V, tbuf, j)

        pltpu.sync_copy(outt_v, outt_hbm.at[rows, :])

    return gather_kernel


def kernel(idx, emb):
    B, = idx.shape
    V, D = emb.shape
    idx32 = idx.astype(jnp.int32)
    sidx, pos = lax.sort_key_val(idx32, lax.iota(jnp.int32, B))
    sidx = jnp.concatenate([sidx, jnp.full((16,), jnp.int32(2**31 - 1))])
    pos = jnp.concatenate([pos, jnp.zeros((16,), jnp.int32)])
    tail = V % 128
    tailt = emb[V - tail:, :].T if tail else jnp.zeros((D, 1), jnp.float32)
    outt = _build(B, V, D)(emb.T, sidx, pos, tailt)
    return outt.T


# speedup vs baseline: 1.2243x; 1.2243x over previous
import functools

import jax
import jax.numpy as jnp
from jax import lax
from jax.experimental import pallas as pl
from jax.experimental.pallas import tpu as pltpu
from jax.experimental.pallas import tpu_sc as plsc

_W = 2048
_NBUF = 8


@functools.lru_cache(maxsize=None)
def _build(B, V, D):
    info = plsc.get_sparse_core_info()
    NC, NS, L = info.num_cores, info.num_subcores, info.num_lanes
    NW = NC * NS
    FPT = D // NW
    n_full = V // _W
    mid = (V - n_full * _W) // 128 * 128
    tail = V - n_full * _W - mid
    n_grp = n_full // _NBUF
    assert FPT >= 1 and _W % 128 == 0 and n_full % _NBUF == 0 and B % L == 0
    mesh = plsc.VectorSubcoreMesh(core_axis_name="c", subcore_axis_name="s")

    @functools.partial(
        pl.kernel,
        mesh=mesh,
        out_type=jax.ShapeDtypeStruct((D, B), jnp.float32),
        scratch_types=[
            pltpu.VMEM((B + L,), jnp.int32),
            pltpu.VMEM((B + L,), jnp.int32),
            [pltpu.VMEM((FPT, _W), jnp.float32)] * _NBUF,
            pltpu.VMEM((FPT, max(tail, 1)), jnp.float32),
            pltpu.VMEM((FPT, B), jnp.float32),
            [pltpu.SemaphoreType.DMA] * _NBUF,
        ],
        compiler_params=pltpu.CompilerParams(needs_layout_passes=False),
    )
    def gather_kernel(embt_hbm, sidx_hbm, pos_hbm, tailt_hbm, outt_hbm,
                      sidx_v, pos_v, bufs, tbuf, outt_v, sems):
        wid = lax.axis_index("s") * NC + lax.axis_index("c")
        f0 = wid * FPT
        rows = pl.ds(f0, FPT)
        pltpu.sync_copy(sidx_hbm, sidx_v)
        pltpu.sync_copy(pos_hbm, pos_v)

        def proc_block(n0, n1, buf, j):
            def cond(carry):
                return carry[1]

            def step(carry):
                j, _ = carry
                v = sidx_v[pl.ds(j, L)]
                mask = v < n1
                local = v - n0
                p = pos_v[pl.ds(j, L)]
                for f in range(FPT):
                    fs = jnp.full((L,), 0, jnp.int32) + f
                    vals = plsc.load_gather(buf, [fs, local], mask=mask)
                    plsc.store_scatter(outt_v, [fs, p], vals, mask=mask)
                cnt = plsc.all_reduce_population_count(mask)[0]
                return j + cnt, cnt == L

            j, _ = lax.while_loop(cond, step, (j, True))
            return j

        def start(k, buf, sem):
            pltpu.async_copy(
                embt_hbm.at[rows, pl.ds(k * _W, _W)], buf, sem)

        def wait(k, buf, sem):
            pltpu.make_async_copy(
                embt_hbm.at[rows, pl.ds(k * _W, _W)], buf, sem).wait()

        for u in range(_NBUF):
            start(u, bufs[u], sems[u])

        def grp_body(i, j):
            k0 = _NBUF * i
            for u in range(_NBUF):
                k = k0 + u
                wait(k, bufs[u], sems[u])
                j = proc_block(k * _W, (k + 1) * _W, bufs[u], j)

                @pl.when(i < n_grp - 1)
                def _():
                    start(k + _NBUF, bufs[u], sems[u])

            return j

        j = lax.fori_loop(0, n_grp, grp_body, 0)

        if mid:
            mbuf = bufs[0].at[:, pl.ds(0, mid)]
            pltpu.sync_copy(embt_hbm.at[rows, pl.ds(n_full * _W, mid)], mbuf)
            j = proc_block(n_full * _W, n_full * _W + mid, mbuf, j)

        if tail:
            pltpu.sync_copy(tailt_hbm.at[rows, :], tbuf)
            proc_block(V - tail, V, tbuf, j)

        pltpu.sync_copy(outt_v, outt_hbm.at[rows, :])

    return gather_kernel


def kernel(idx, emb):
    B, = idx.shape
    V, D = emb.shape
    idx32 = idx.astype(jnp.int32)
    sidx, pos = lax.sort_key_val(idx32, lax.iota(jnp.int32, B))
    sidx = jnp.concatenate([sidx, jnp.full((16,), jnp.int32(2**31 - 1))])
    pos = jnp.concatenate([pos, jnp.zeros((16,), jnp.int32)])
    tail = V % 128
    tailt = emb[V - tail:, :].T if tail else jnp.zeros((D, 1), jnp.float32)
    outt = _build(B, V, D)(emb.T, sidx, pos, tailt)
    return outt.T
